# baseline (speedup 1.0000x reference)
import dataclasses

import jax
import jax.numpy as jnp
from jax import lax
from jax.experimental import pallas as pl
from jax.experimental.pallas import tpu as pltpu
from jax.experimental.pallas import tpu_sc as plsc

_LANES = 16
_TILES = 32
_CHUNK = 32
_SC_BATCHES = 16


def _stats_core(x):
    l, d = x.shape
    xb = x.astype(jnp.bfloat16)
    ones = jnp.ones((8, l), jnp.bfloat16)
    px8 = jax.lax.dot(ones, xb, preferred_element_type=jnp.float32)
    px = px8[0:1]
    rng = jax.lax.broadcasted_iota(jnp.int32, (1, d), 1).astype(jnp.float32) - d / 2.0
    px = px / jnp.sum(px)
    mean = jnp.sum(px * rng)
    std = jnp.sqrt(jnp.sum(px * (rng - mean) ** 2))
    new_idx = (rng - mean) / std + d / 2.0
    ii = new_idx.astype(jnp.int32)
    fl = jnp.clip(ii, 0, d - 1)
    ce = jnp.clip(ii + 1, 0, d - 1)
    w = new_idx - jnp.floor(new_idx)
    return xb, fl, ce, w


def _fused_body(x_ref, o_ref):
    x = x_ref[0]
    l, d = x.shape
    xb, fl, ce, w = _stats_core(x)
    rows = jax.lax.broadcasted_iota(jnp.int32, (d, d), 0)
    g = jnp.where(rows == fl, 1.0 - w, 0.0) + jnp.where(rows == ce, w, 0.0)
    o_ref[0] = jax.lax.dot(
        xb, g.astype(jnp.bfloat16), preferred_element_type=jnp.float32
    )


def _stats_body(x_ref, fl_ref, ce_ref, w_ref):
    _, fl, ce, w = _stats_core(x_ref[0])
    fl_ref[0] = fl
    ce_ref[0] = ce
    w_ref[0] = w


def _make_sc_gather_body(batch_off, nb, l, d):
    rows_per_tile = l // _TILES
    assert rows_per_tile == 2 * _CHUNK

    def body(x_hbm, fl_hbm, ce_hbm, w_hbm, o_hbm,
             in0, in1, out0, out1, flv, cev, wv,
             isem0, isem1, osem0, osem1):
        wid = lax.axis_index("s") * 2 + lax.axis_index("c")
        rbase = wid * rows_per_tile

        pltpu.sync_copy(fl_hbm, flv)
        pltpu.sync_copy(ce_hbm, cev)
        pltpu.sync_copy(w_hbm, wv)

        def start_in(bi, ch0, buf, sem):
            return pltpu.async_copy(
                x_hbm.at[bi + batch_off, pl.ds(rbase + ch0, _CHUNK)], buf, sem)

        def drain(buf, sem):
            pltpu.make_async_copy(
                x_hbm.at[batch_off, pl.ds(rbase, _CHUNK)], buf, sem).wait()

        def compute(bi, src, dst):
            for c in range(0, d, _LANES):
                f_idx = flv[bi, pl.ds(c, _LANES)]
                c_idx = cev[bi, pl.ds(c, _LANES)]
                wvec = wv[bi, pl.ds(c, _LANES)]

                @pl.loop(0, _CHUNK, step=4)
                def _row(r):
                    rv0 = jnp.full((_LANES,), 0, jnp.int32) + r
                    rv1 = rv0 + 1
                    rv2 = rv0 + 2
                    rv3 = rv0 + 3
                    gf0 = plsc.load_gather(src, [rv0, f_idx])
                    gc0 = plsc.load_gather(src, [rv0, c_idx])
                    gf1 = plsc.load_gather(src, [rv1, f_idx])
                    gc1 = plsc.load_gather(src, [rv1, c_idx])
                    gf2 = plsc.load_gather(src, [rv2, f_idx])
                    gc2 = plsc.load_gather(src, [rv2, c_idx])
                    gf3 = plsc.load_gather(src, [rv3, f_idx])
                    gc3 = plsc.load_gather(src, [rv3, c_idx])
                    dst[r, pl.ds(c, _LANES)] = gf0 + wvec * (gc0 - gf0)
                    dst[r + 1, pl.ds(c, _LANES)] = gf1 + wvec * (gc1 - gf1)
                    dst[r + 2, pl.ds(c, _LANES)] = gf2 + wvec * (gc2 - gf2)
                    dst[r + 3, pl.ds(c, _LANES)] = gf3 + wvec * (gc3 - gf3)

        start_in(0, 0, in0, isem0)
        start_in(0, _CHUNK, in1, isem1)

        @pl.loop(0, nb)
        def _batch(bi):
            drain(in0, isem0)

            @pl.when(bi > 0)
            def _():
                drain(out0, osem0)

            compute(bi, in0, out0)
            pltpu.async_copy(out0, o_hbm.at[bi, pl.ds(rbase, _CHUNK)], osem0)

            @pl.when(bi + 1 < nb)
            def _():
                start_in(bi + 1, 0, in0, isem0)

            drain(in1, isem1)

            @pl.when(bi > 0)
            def _():
                drain(out1, osem1)

            compute(bi, in1, out1)
            pltpu.async_copy(
                out1, o_hbm.at[bi, pl.ds(rbase + _CHUNK, _CHUNK)], osem1)

            @pl.when(bi + 1 < nb)
            def _():
                start_in(bi + 1, _CHUNK, in1, isem1)

        drain(out0, osem0)
        drain(out1, osem1)

    return body


def kernel(distance):
    b, l, d = distance.shape
    k = _SC_BATCHES
    bt = b - k
    i32 = jnp.int32

    fl, ce, w = pl.pallas_call(
        _stats_body,
        grid=(k,),
        in_specs=[pl.BlockSpec((1, l, d), lambda i: (i + bt, 0, 0))],
        out_specs=[
            pl.BlockSpec((1, 1, d), lambda i: (i, 0, 0)),
            pl.BlockSpec((1, 1, d), lambda i: (i, 0, 0)),
            pl.BlockSpec((1, 1, d), lambda i: (i, 0, 0)),
        ],
        out_shape=[
            jax.ShapeDtypeStruct((k, 1, d), i32),
            jax.ShapeDtypeStruct((k, 1, d), i32),
            jax.ShapeDtypeStruct((k, 1, d), jnp.float32),
        ],
    )(distance)
    fl, ce, w = fl.reshape(k, d), ce.reshape(k, d), w.reshape(k, d)

    mesh = plsc.VectorSubcoreMesh(core_axis_name="c", subcore_axis_name="s")
    cp = pltpu.CompilerParams()
    if "needs_layout_passes" in pltpu.CompilerParams.__dataclass_fields__:
        cp = dataclasses.replace(cp, needs_layout_passes=False)
    sc_out = pl.kernel(
        _make_sc_gather_body(bt, k, l, d),
        out_type=jax.ShapeDtypeStruct((k, l, d), jnp.float32),
        mesh=mesh,
        scratch_types=[
            pltpu.VMEM((_CHUNK, d), jnp.float32),
            pltpu.VMEM((_CHUNK, d), jnp.float32),
            pltpu.VMEM((_CHUNK, d), jnp.float32),
            pltpu.VMEM((_CHUNK, d), jnp.float32),
            pltpu.VMEM((k, d), i32),
            pltpu.VMEM((k, d), i32),
            pltpu.VMEM((k, d), jnp.float32),
            pltpu.SemaphoreType.DMA,
            pltpu.SemaphoreType.DMA,
            pltpu.SemaphoreType.DMA,
            pltpu.SemaphoreType.DMA,
        ],
        compiler_params=cp,
    )(distance, fl, ce, w)

    if bt == 0:
        return sc_out

    tc_out = pl.pallas_call(
        _fused_body,
        grid=(bt,),
        in_specs=[pl.BlockSpec((1, l, d), lambda i: (i, 0, 0))],
        out_specs=pl.BlockSpec((1, l, d), lambda i: (i, 0, 0)),
        out_shape=jax.ShapeDtypeStruct((bt, l, d), distance.dtype),
    )(distance)

    return jnp.concatenate([tc_out, sc_out], axis=0)


# speedup vs baseline: 4.1222x; 4.1222x over previous
import jax
import jax.numpy as jnp
from jax.experimental import pallas as pl


def _body(x_ref, o_ref):
    x = x_ref[0]
    L, D = x.shape
    rng = jax.lax.broadcasted_iota(jnp.int32, (1, D), 1).astype(jnp.float32) - D / 2.0
    px = jnp.sum(x, axis=0, keepdims=True)
    px = px / jnp.sum(px)
    mean = jnp.sum(px * rng)
    std = jnp.sqrt(jnp.sum(px * (rng - mean) ** 2))
    new_idx = (rng - mean) / std + D / 2.0
    ii = new_idx.astype(jnp.int32)
    fl = jnp.clip(ii, 0, D - 1)
    ce = jnp.clip(ii + 1, 0, D - 1)
    w = new_idx - jnp.floor(new_idx)
    rows = jax.lax.broadcasted_iota(jnp.int32, (D, D), 0)
    g = jnp.where(rows == fl, 1.0 - w, 0.0) + jnp.where(rows == ce, w, 0.0)
    o_ref[0] = jax.lax.dot(
        x.astype(jnp.bfloat16),
        g.astype(jnp.bfloat16),
        preferred_element_type=jnp.float32,
    )


def kernel(distance):
    b, l, d = distance.shape
    return pl.pallas_call(
        _body,
        grid=(b,),
        in_specs=[pl.BlockSpec((1, l, d), lambda i: (i, 0, 0))],
        out_specs=pl.BlockSpec((1, l, d), lambda i: (i, 0, 0)),
        out_shape=jax.ShapeDtypeStruct((b, l, d), distance.dtype),
    )(distance)


# speedup vs baseline: 4.1289x; 1.0016x over previous
import jax
import jax.numpy as jnp
from jax.experimental import pallas as pl


def _body(xa_ref, xb_ref, o_ref):
    xa = xa_ref[0]
    xb = xb_ref[0]
    D = xa.shape[1]
    x = jnp.concatenate([xa, xb], axis=0)
    rng = jax.lax.broadcasted_iota(jnp.int32, (1, D), 1).astype(jnp.float32) - D / 2.0
    px = jnp.sum(x, axis=0, keepdims=True)
    px = px / jnp.sum(px)
    mean = jnp.sum(px * rng)
    std = jnp.sqrt(jnp.sum(px * (rng - mean) ** 2))
    new_idx = (rng - mean) / std + D / 2.0
    ii = new_idx.astype(jnp.int32)
    fl = jnp.clip(ii, 0, D - 1)
    ce = jnp.clip(ii + 1, 0, D - 1)
    w = new_idx - jnp.floor(new_idx)
    rows = jax.lax.broadcasted_iota(jnp.int32, (D, D), 0)
    g = jnp.where(rows == fl, 1.0 - w, 0.0) + jnp.where(rows == ce, w, 0.0)
    o_ref[0] = jax.lax.dot(
        x.astype(jnp.bfloat16),
        g.astype(jnp.bfloat16),
        preferred_element_type=jnp.float32,
    )


def kernel(distance):
    b, l, d = distance.shape
    h = l // 2
    return pl.pallas_call(
        _body,
        grid=(b,),
        in_specs=[
            pl.BlockSpec((1, h, d), lambda i: (i, 0, 0)),
            pl.BlockSpec((1, h, d), lambda i: (i, 1, 0)),
        ],
        out_specs=pl.BlockSpec((1, l, d), lambda i: (i, 0, 0)),
        out_shape=jax.ShapeDtypeStruct((b, l, d), distance.dtype),
    )(distance, distance)
